# baseline (speedup 1.0000x reference)
import functools

import jax
import jax.numpy as jnp
from jax import lax
from jax.experimental import pallas as pl
from jax.experimental.pallas import tpu as pltpu
from jax.experimental.pallas import tpu_sc as plsc

IDXW = 128


def _make_gather(vocab, d_model, num_idx):
    info = plsc.get_sparse_core_info()
    nc, ns = info.num_cores, info.num_subcores
    nw = nc * ns
    per_w = num_idx // nw
    k = 4
    chunk = k * IDXW
    n_chunks = per_w // chunk
    assert per_w % chunk == 0

    mesh = plsc.VectorSubcoreMesh(core_axis_name="c", subcore_axis_name="s")

    @functools.partial(
        pl.kernel,
        mesh=mesh,
        compiler_params=pltpu.CompilerParams(use_tc_tiling_on_sc=False),
        out_type=jax.ShapeDtypeStruct((num_idx, d_model), jnp.float32),
        scratch_types=[
            pltpu.VMEM((k, IDXW), jnp.int32),
            pltpu.VMEM((chunk, d_model), jnp.float32),
            pltpu.SemaphoreType.DMA,
        ],
    )
    def gather_kernel(idx_hbm, table_hbm, out_hbm, idx_v, rows_v, sem):
        wid = lax.axis_index("s") * nc + lax.axis_index("c")
        row0 = wid * (per_w // IDXW)

        def body(i, carry):
            r = row0 + i * k
            pltpu.sync_copy(idx_hbm.at[pl.ds(r, k)], idx_v)
            copies = [
                pltpu.async_copy(
                    table_hbm.at[idx_v.at[j]],
                    rows_v.at[pl.ds(j * IDXW, IDXW)],
                    sem,
                )
                for j in range(k)
            ]
            for c in copies:
                c.wait()
            pltpu.sync_copy(rows_v, out_hbm.at[pl.ds(r * IDXW, chunk)])
            return carry

        lax.fori_loop(0, n_chunks, body, 0)

    return gather_kernel


def kernel(indices, table):
    b, s = indices.shape
    vocab, d_model = table.shape
    num_idx = b * s
    idx2d = indices.reshape(num_idx // IDXW, IDXW).astype(jnp.int32)
    out = _make_gather(vocab, d_model, num_idx)(idx2d, table)
    return out.reshape(b, s, d_model)


# speedup vs baseline: 1.0446x; 1.0446x over previous
import functools

import jax
import jax.numpy as jnp
from jax import lax
from jax.experimental import pallas as pl
from jax.experimental.pallas import tpu as pltpu
from jax.experimental.pallas import tpu_sc as plsc

IDXW = 128
NBUF = 3


def _make_gather(vocab, d_model, num_idx):
    info = plsc.get_sparse_core_info()
    nc, ns = info.num_cores, info.num_subcores
    nw = nc * ns
    per_w = num_idx // nw
    k = 4
    chunk = k * IDXW
    n_chunks = per_w // chunk
    idx_rows = per_w // IDXW
    assert per_w % chunk == 0 and n_chunks > NBUF

    mesh = plsc.VectorSubcoreMesh(core_axis_name="c", subcore_axis_name="s")

    @functools.partial(
        pl.kernel,
        mesh=mesh,
        compiler_params=pltpu.CompilerParams(use_tc_tiling_on_sc=False),
        out_type=jax.ShapeDtypeStruct((num_idx, d_model), jnp.float32),
        scratch_types=[
            pltpu.VMEM((idx_rows, IDXW), jnp.int32),
            [pltpu.VMEM((chunk, d_model), jnp.float32) for _ in range(NBUF)],
            [pltpu.SemaphoreType.DMA for _ in range(NBUF)],
            [pltpu.SemaphoreType.DMA for _ in range(NBUF)],
        ],
    )
    def gather_kernel(idx_hbm, table_hbm, out_hbm, idx_v, rows, sem_g, sem_st):
        wid = lax.axis_index("s") * nc + lax.axis_index("c")
        row0 = wid * idx_rows
        base = wid * per_w

        def fire_gathers(c, j):
            for q in range(k):
                pltpu.async_copy(
                    table_hbm.at[idx_v.at[c * k + q]],
                    rows[j].at[pl.ds(q * IDXW, IDXW)],
                    sem_g[j],
                )

        def wait_gathers(j):
            pltpu.make_async_copy(
                table_hbm.at[pl.ds(0, chunk)], rows[j], sem_g[j]
            ).wait()

        def fire_store(c, j):
            pltpu.async_copy(
                rows[j], out_hbm.at[pl.ds(base + c * chunk, chunk)], sem_st[j]
            )

        def wait_store(j):
            pltpu.make_async_copy(
                rows[j], out_hbm.at[pl.ds(0, chunk)], sem_st[j]
            ).wait()

        def steady(c, j):
            wait_store(j)
            fire_gathers(c, j)
            wait_gathers((j + 1) % NBUF)
            fire_store(c - 2, (j + 1) % NBUF)

        pltpu.sync_copy(idx_hbm.at[pl.ds(row0, idx_rows)], idx_v)
        for c in range(NBUF):
            fire_gathers(c, c)
        wait_gathers(0)
        fire_store(0, 0)

        groups = (n_chunks - NBUF) // NBUF
        tail0 = NBUF + groups * NBUF

        def body(i, carry):
            g = NBUF + i * NBUF
            for j in range(NBUF):
                steady(g + j, j)
            return carry

        lax.fori_loop(0, groups, body, 0)

        for c in range(tail0, n_chunks):
            steady(c, c % NBUF)
        for c in range(n_chunks - 2, n_chunks):
            wait_gathers(c % NBUF)
            fire_store(c, c % NBUF)
        for c in range(n_chunks - NBUF, n_chunks):
            wait_store(c % NBUF)

    return gather_kernel


def kernel(indices, table):
    b, s = indices.shape
    vocab, d_model = table.shape
    num_idx = b * s
    idx2d = indices.reshape(num_idx // IDXW, IDXW).astype(jnp.int32)
    out = _make_gather(vocab, d_model, num_idx)(idx2d, table)
    return out.reshape(b, s, d_model)
